# baseline (speedup 1.0000x reference)
import functools

import jax
import jax.numpy as jnp
from jax import lax
from jax.experimental import pallas as pl
from jax.experimental.pallas import tpu as pltpu
from jax.experimental.pallas import tpu_sc as plsc

_NC, _NS = 2, 16
_NW = _NC * _NS
_NBG = 8
_NTQ = 4
_NBUF = 4
_LANES = 16
_HROWS = 16


def _sc_add(inputs, pos_table):
    B, T, D = inputs.shape
    BPG = B // _NBG
    TQ = T // _NTQ
    NCH = BPG * (TQ // _HROWS)

    mesh = plsc.VectorSubcoreMesh(core_axis_name="c", subcore_axis_name="s")

    @functools.partial(
        pl.kernel,
        out_type=jax.ShapeDtypeStruct((B, T, D), inputs.dtype),
        mesh=mesh,
        scratch_types=[
            pltpu.VMEM((TQ, D), jnp.float32),
            [pltpu.VMEM((_HROWS, D), jnp.float32) for _ in range(_NBUF)],
            [pltpu.SemaphoreType.DMA for _ in range(_NBUF)],
            [pltpu.SemaphoreType.DMA for _ in range(_NBUF)],
        ],
    )
    def k(in_hbm, tab_hbm, out_hbm, tab_v, bufs, isems, osems):
        wid = lax.axis_index("s") * _NC + lax.axis_index("c")
        b0 = (wid % _NBG) * BPG
        t0 = (wid // _NBG) * TQ
        pltpu.sync_copy(tab_hbm.at[pl.ds(t0, TQ), :], tab_v)

        def chunk_slice(ref, g):
            return ref.at[b0 + g // 2, pl.ds(t0 + (g % 2) * _HROWS, _HROWS), :]

        def start_in(g, p):
            pltpu.async_copy(chunk_slice(in_hbm, g), bufs[p], isems[p])

        def start_out(g, p):
            pltpu.async_copy(bufs[p], chunk_slice(out_hbm, g), osems[p])

        def wait_in(p):
            pltpu.make_async_copy(chunk_slice(in_hbm, 0), bufs[p], isems[p]).wait()

        def wait_out(p):
            pltpu.make_async_copy(bufs[p], chunk_slice(out_hbm, 0), osems[p]).wait()

        def compute(g, p):
            buf = bufs[p]
            toff = (g % 2) * _HROWS

            def row(r, carry):
                for kk in range(D // _LANES):
                    sl = pl.ds(kk * _LANES, _LANES)
                    plsc.addupdate(buf.at[r, sl], tab_v[toff + r, sl])
                return carry

            lax.fori_loop(0, _HROWS, row, 0)

        def step(g, p, issue_in, first):
            if issue_in:
                q = (p + 2) % _NBUF
                if not first:
                    wait_out(q)
                start_in(g + 2, q)
            wait_in(p)
            compute(g, p)
            start_out(g, p)

        start_in(0, 0)
        start_in(1, 1)

        for p in range(_NBUF):
            step(p, p, True, first=(p < 2))

        def body(h, carry):
            g = h * _NBUF
            for p in range(_NBUF):
                step(g + p, p, True, False)
            return carry

        lax.fori_loop(1, NCH // _NBUF - 1, body, 0)

        gt = NCH - _NBUF
        for p in range(_NBUF):
            step(gt + p, p, issue_in=(p < 2), first=False)

        for p in range(_NBUF):
            wait_out(p)

    return k(inputs, pos_table)


def kernel(inputs, pos_table):
    return _sc_add(inputs, pos_table)


# speedup vs baseline: 2.3059x; 2.3059x over previous
import functools

import jax
import jax.numpy as jnp
from jax import lax
from jax.experimental import pallas as pl
from jax.experimental.pallas import tpu as pltpu
from jax.experimental.pallas import tpu_sc as plsc

_NC, _NS = 2, 16
_NW = _NC * _NS
_NBG = 8
_NTQ = 4
_NBUF = 4
_LANES = 16
_HROWS = 16


def _sc_add(inputs, pos_table):
    B, T, D = inputs.shape
    BPG = B // _NBG
    TQ = T // _NTQ
    NCH = BPG * (TQ // _HROWS)

    mesh = plsc.VectorSubcoreMesh(core_axis_name="c", subcore_axis_name="s")

    @functools.partial(
        pl.kernel,
        out_type=jax.ShapeDtypeStruct((B, T, D), inputs.dtype),
        mesh=mesh,
        scratch_types=[
            pltpu.VMEM((TQ, D), jnp.float32),
            [pltpu.VMEM((_HROWS, D), jnp.float32) for _ in range(_NBUF)],
            [pltpu.SemaphoreType.DMA for _ in range(_NBUF)],
            [pltpu.SemaphoreType.DMA for _ in range(_NBUF)],
        ],
    )
    def k(in_hbm, tab_hbm, out_hbm, tab_v, bufs, isems, osems):
        wid = lax.axis_index("s") * _NC + lax.axis_index("c")
        b0 = (wid % _NBG) * BPG
        t0 = (wid // _NBG) * TQ
        pltpu.sync_copy(tab_hbm.at[pl.ds(t0, TQ), :], tab_v)

        def chunk_slice(ref, g):
            return ref.at[b0 + g // 2, pl.ds(t0 + (g % 2) * _HROWS, _HROWS), :]

        def start_in(g, p):
            pltpu.async_copy(chunk_slice(in_hbm, g), bufs[p], isems[p])

        def start_out(g, p):
            pltpu.async_copy(bufs[p], chunk_slice(out_hbm, g), osems[p])

        def wait_in(p):
            pltpu.make_async_copy(chunk_slice(in_hbm, 0), bufs[p], isems[p]).wait()

        def wait_out(p):
            pltpu.make_async_copy(bufs[p], chunk_slice(out_hbm, 0), osems[p]).wait()

        def compute(g, p):
            buf = bufs[p]
            toff = (g % 2) * _HROWS
            kpr = D // _LANES

            @plsc.parallel_loop(0, _HROWS * kpr, unroll=8)
            def _(j):
                r = j // kpr
                kk = j % kpr
                sl = pl.ds(kk * _LANES, _LANES)
                plsc.addupdate(buf.at[r, sl], tab_v[toff + r, sl])

        def step(g, p, issue_in, first):
            if issue_in:
                q = (p + 2) % _NBUF
                if not first:
                    wait_out(q)
                start_in(g + 2, q)
            wait_in(p)
            compute(g, p)
            start_out(g, p)

        start_in(0, 0)
        start_in(1, 1)

        for p in range(_NBUF):
            step(p, p, True, first=(p < 2))

        def body(h, carry):
            g = h * _NBUF
            for p in range(_NBUF):
                step(g + p, p, True, False)
            return carry

        lax.fori_loop(1, NCH // _NBUF - 1, body, 0)

        gt = NCH - _NBUF
        for p in range(_NBUF):
            step(gt + p, p, issue_in=(p < 2), first=False)

        for p in range(_NBUF):
            wait_out(p)

    return k(inputs, pos_table)


def kernel(inputs, pos_table):
    return _sc_add(inputs, pos_table)
